# baseline (speedup 1.0000x reference)
import functools

import jax
import jax.numpy as jnp
from jax import lax
from jax.experimental import pallas as pl
from jax.experimental.pallas import tpu as pltpu
from jax.experimental.pallas import tpu_sc as plsc

NUM_WORKERS = 32
LANES = 16
CHUNK = 16
SBLK = 512
SC_TOKENS = 14336


def _build_sc(num_tokens: int, hidden: int):
    per_worker = num_tokens // NUM_WORKERS
    ngroups = per_worker // LANES
    mesh = plsc.VectorSubcoreMesh(core_axis_name="c", subcore_axis_name="s")

    @functools.partial(
        pl.kernel,
        out_type=jax.ShapeDtypeStruct((num_tokens, hidden), jnp.float32),
        mesh=mesh,
        compiler_params=pltpu.CompilerParams(needs_layout_passes=False),
        scratch_types=[
            pltpu.VMEM((per_worker,), jnp.int32),
            pltpu.VMEM((CHUNK, hidden), jnp.float32),
            pltpu.VMEM((CHUNK, hidden), jnp.float32),
            pltpu.VMEM((per_worker + LANES,), jnp.int32),
            pltpu.VMEM((per_worker + LANES,), jnp.int32),
            pltpu.VMEM((ngroups + 1, LANES), jnp.int32),
            pltpu.VMEM((ngroups + 1, LANES), jnp.int32),
            pltpu.VMEM((2, hidden), jnp.float32),
            pltpu.SemaphoreType.DMA,
        ],
    )
    def run(ids_hbm, w_hbm, out_hbm, ids_v, c0_v, c1_v, pf0, pf1,
            p2d0, p2d1, w_v, sem):
        cid = lax.axis_index("c")
        sid = lax.axis_index("s")
        wid = sid * 2 + cid
        base = wid * per_worker
        lanes = lax.iota(jnp.int32, LANES)
        pltpu.sync_copy(w_hbm, w_v)
        pltpu.sync_copy(ids_hbm.at[wid], ids_v)

        def fill(j, carry):
            v0 = w_v[0, pl.ds(j * LANES, LANES)]
            v1 = w_v[1, pl.ds(j * LANES, LANES)]
            for r in range(CHUNK):
                c0_v[r, pl.ds(j * LANES, LANES)] = v0
                c1_v[r, pl.ds(j * LANES, LANES)] = v1
            return carry

        lax.fori_loop(0, hidden // LANES, fill, 0)

        def group(g, carry):
            cur0, cur1 = carry
            ids_vec = ids_v[pl.ds(g * LANES, LANES)]
            positions = base + g * LANES + lanes
            m0 = ids_vec == 0
            m1 = jnp.logical_not(m0)
            plsc.store_compressed(pf0.at[pl.ds(cur0, LANES)], positions, mask=m0)
            plsc.store_compressed(pf1.at[pl.ds(cur1, LANES)], positions, mask=m1)
            n0v = plsc.all_reduce_population_count(m0)
            n0 = n0v if n0v.ndim == 0 else n0v[0]
            return (cur0 + n0, cur1 + (LANES - n0))

        cur0, cur1 = lax.fori_loop(
            0, ngroups, group, (jnp.int32(0), jnp.int32(0)))

        zero16 = lanes * 0
        pf0[pl.ds(cur0, LANES)] = plsc.load_gather(pf0, [zero16])
        pf1[pl.ds(cur1, LANES)] = plsc.load_gather(pf1, [zero16])

        def stage(k, carry):
            p2d0[k, :] = pf0[pl.ds(k * LANES, LANES)]
            p2d1[k, :] = pf1[pl.ds(k * LANES, LANES)]
            return carry

        lax.fori_loop(0, ngroups + 1, stage, 0)

        nc0 = (cur0 + (CHUNK - 1)) // CHUNK
        nc1 = (cur1 + (CHUNK - 1)) // CHUNK

        def fire0(k, carry):
            pltpu.async_copy(c0_v, out_hbm.at[p2d0.at[k]], sem)
            return carry

        def fire1(k, carry):
            pltpu.async_copy(c1_v, out_hbm.at[p2d1.at[k]], sem)
            return carry

        lax.fori_loop(0, nc0, fire0, 0)
        lax.fori_loop(0, nc1, fire1, 0)

        def drain(k, carry):
            pltpu.make_async_copy(c0_v, out_hbm.at[p2d0.at[0]], sem).wait()
            return carry

        lax.fori_loop(0, nc0 + nc1, drain, 0)

    return run


def _tc_select(ids3, weight, hidden):
    nblk = ids3.shape[0]

    def body(ids_ref, w_ref, out_ref):
        idv = ids_ref[0, 0, :].astype(jnp.float32)[:, None]
        w0 = w_ref[0, :][None, :]
        w1 = w_ref[1, :][None, :]
        out_ref[...] = w0 + idv * (w1 - w0)

    return pl.pallas_call(
        body,
        grid=(nblk,),
        in_specs=[
            pl.BlockSpec((1, 1, SBLK), lambda i: (i, 0, 0)),
            pl.BlockSpec((2, hidden), lambda i: (0, 0)),
        ],
        out_specs=pl.BlockSpec((SBLK, hidden), lambda i: (i, 0)),
        out_shape=jax.ShapeDtypeStruct((nblk * SBLK, hidden), jnp.float32),
    )(ids3, weight)


def kernel(tokentype_ids, weight):
    batch, seq = tokentype_ids.shape
    vocab, hidden = weight.shape
    num_tokens = batch * seq
    ids_flat = tokentype_ids.reshape(num_tokens)

    n_sc = SC_TOKENS
    ids_sc = ids_flat[:n_sc].reshape(NUM_WORKERS, n_sc // NUM_WORKERS)
    ids_tc = ids_flat[n_sc:].reshape((num_tokens - n_sc) // SBLK, 1, SBLK)

    out_sc = _build_sc(n_sc, hidden)(ids_sc, weight)
    out_tc = _tc_select(ids_tc, weight, hidden)
    out = jnp.concatenate([out_sc, out_tc], axis=0)
    return out.reshape(batch, seq, hidden)


# speedup vs baseline: 2.4725x; 2.4725x over previous
import functools

import jax
import jax.numpy as jnp
from jax import lax
from jax.experimental import pallas as pl
from jax.experimental.pallas import tpu as pltpu
from jax.experimental.pallas import tpu_sc as plsc

NUM_WORKERS = 32
LANES = 16
CHUNK = 16


def _build(num_tokens: int, hidden: int):
    per_worker = num_tokens // NUM_WORKERS
    ngroups = per_worker // LANES
    mesh = plsc.VectorSubcoreMesh(core_axis_name="c", subcore_axis_name="s")

    @functools.partial(
        pl.kernel,
        out_type=jax.ShapeDtypeStruct((num_tokens, hidden), jnp.float32),
        mesh=mesh,
        compiler_params=pltpu.CompilerParams(needs_layout_passes=False),
        scratch_types=[
            pltpu.VMEM((per_worker,), jnp.int32),
            pltpu.VMEM((CHUNK, hidden), jnp.float32),
            pltpu.VMEM((CHUNK, hidden), jnp.float32),
            pltpu.VMEM((per_worker + LANES,), jnp.int32),
            pltpu.VMEM((per_worker + LANES,), jnp.int32),
            pltpu.VMEM((ngroups + 1, LANES), jnp.int32),
            pltpu.VMEM((ngroups + 1, LANES), jnp.int32),
            pltpu.VMEM((2, hidden), jnp.float32),
            pltpu.SemaphoreType.DMA,
        ],
    )
    def run(ids_hbm, w_hbm, out_hbm, ids_v, c0_v, c1_v, pf0, pf1,
            p2d0, p2d1, w_v, sem):
        cid = lax.axis_index("c")
        sid = lax.axis_index("s")
        wid = sid * 2 + cid
        base = wid * per_worker
        lanes = lax.iota(jnp.int32, LANES)
        pltpu.sync_copy(w_hbm, w_v)
        pltpu.sync_copy(ids_hbm.at[wid], ids_v)

        def group(g, carry):
            cur0, cur1 = carry
            ids_vec = ids_v[pl.ds(g * LANES, LANES)]
            positions = base + g * LANES + lanes
            m0 = ids_vec == 0
            m1 = jnp.logical_not(m0)
            plsc.store_compressed(pf0.at[pl.ds(cur0, LANES)], positions, mask=m0)
            plsc.store_compressed(pf1.at[pl.ds(cur1, LANES)], positions, mask=m1)
            n0v = plsc.all_reduce_population_count(m0)
            n0 = n0v if n0v.ndim == 0 else n0v[0]
            return (cur0 + n0, cur1 + (LANES - n0))

        cur0, cur1 = lax.fori_loop(
            0, ngroups, group, (jnp.int32(0), jnp.int32(0)))

        zero16 = lanes * 0
        pf0[pl.ds(cur0, LANES)] = plsc.load_gather(pf0, [zero16])
        pf1[pl.ds(cur1, LANES)] = plsc.load_gather(pf1, [zero16])

        def stage(k, carry):
            p2d0[k, :] = pf0[pl.ds(k * LANES, LANES)]
            p2d1[k, :] = pf1[pl.ds(k * LANES, LANES)]
            return carry

        lax.fori_loop(0, ngroups + 1, stage, 0)

        nc0 = (cur0 + (CHUNK - 1)) // CHUNK
        nc1 = (cur1 + (CHUNK - 1)) // CHUNK

        def fire0(k, carry):
            pltpu.async_copy(c0_v, out_hbm.at[p2d0.at[k]], sem)
            return carry

        def fire1(k, carry):
            pltpu.async_copy(c1_v, out_hbm.at[p2d1.at[k]], sem)
            return carry

        def fill0(j, carry):
            v0 = w_v[0, pl.ds(j * LANES, LANES)]
            for r in range(CHUNK):
                c0_v[r, pl.ds(j * LANES, LANES)] = v0
            return carry

        def fill1(j, carry):
            v1 = w_v[1, pl.ds(j * LANES, LANES)]
            for r in range(CHUNK):
                c1_v[r, pl.ds(j * LANES, LANES)] = v1
            return carry

        lax.fori_loop(0, hidden // LANES, fill0, 0)
        lax.fori_loop(0, nc0, fire0, 0)
        lax.fori_loop(0, hidden // LANES, fill1, 0)
        lax.fori_loop(0, nc1, fire1, 0)

        def drain(k, carry):
            pltpu.make_async_copy(c0_v, out_hbm.at[p2d0.at[0]], sem).wait()
            return carry

        lax.fori_loop(0, nc0 + nc1, drain, 0)

    return run


def kernel(tokentype_ids, weight):
    batch, seq = tokentype_ids.shape
    vocab, hidden = weight.shape
    num_tokens = batch * seq
    ids2 = tokentype_ids.reshape(NUM_WORKERS, num_tokens // NUM_WORKERS)
    out = _build(num_tokens, hidden)(ids2, weight)
    return out.reshape(batch, seq, hidden)


# speedup vs baseline: 2.4813x; 1.0036x over previous
import functools

import jax
import jax.numpy as jnp
from jax import lax
from jax.experimental import pallas as pl
from jax.experimental.pallas import tpu as pltpu
from jax.experimental.pallas import tpu_sc as plsc

NUM_WORKERS = 32
LANES = 16
CHUNK = 16


def _build(num_tokens: int, hidden: int):
    per_worker = num_tokens // NUM_WORKERS
    ngroups = per_worker // LANES
    mesh = plsc.VectorSubcoreMesh(core_axis_name="c", subcore_axis_name="s")

    @functools.partial(
        pl.kernel,
        out_type=jax.ShapeDtypeStruct((num_tokens, hidden), jnp.float32),
        mesh=mesh,
        compiler_params=pltpu.CompilerParams(needs_layout_passes=False),
        scratch_types=[
            pltpu.VMEM((per_worker,), jnp.int32),
            pltpu.VMEM((CHUNK, hidden), jnp.float32),
            pltpu.VMEM((CHUNK, hidden), jnp.float32),
            pltpu.VMEM((per_worker + LANES,), jnp.int32),
            pltpu.VMEM((per_worker + LANES,), jnp.int32),
            pltpu.VMEM((ngroups + 1, LANES), jnp.int32),
            pltpu.VMEM((ngroups + 1, LANES), jnp.int32),
            pltpu.VMEM((2, hidden), jnp.float32),
            pltpu.SemaphoreType.DMA,
            pltpu.SemaphoreType.DMA,
        ],
    )
    def run(ids_hbm, w_hbm, out_hbm, ids_v, c0_v, c1_v, pf0, pf1,
            p2d0, p2d1, w_v, sem, sem_ids):
        cid = lax.axis_index("c")
        sid = lax.axis_index("s")
        wid = sid * 2 + cid
        base = wid * per_worker
        lanes = lax.iota(jnp.int32, LANES)
        pltpu.sync_copy(w_hbm, w_v)
        d_ids = pltpu.async_copy(ids_hbm.at[wid], ids_v, sem_ids)

        def fill0(j, carry):
            v0 = w_v[0, pl.ds(j * LANES, LANES)]
            for r in range(CHUNK):
                c0_v[r, pl.ds(j * LANES, LANES)] = v0
            return carry

        lax.fori_loop(0, hidden // LANES, fill0, 0)
        d_ids.wait()

        def group(g, carry):
            cur0, cur1 = carry
            ids_vec = ids_v[pl.ds(g * LANES, LANES)]
            positions = base + g * LANES + lanes
            m0 = ids_vec == 0
            m1 = jnp.logical_not(m0)
            plsc.store_compressed(pf0.at[pl.ds(cur0, LANES)], positions, mask=m0)
            plsc.store_compressed(pf1.at[pl.ds(cur1, LANES)], positions, mask=m1)
            n0v = plsc.all_reduce_population_count(m0)
            n0 = n0v if n0v.ndim == 0 else n0v[0]
            return (cur0 + n0, cur1 + (LANES - n0))

        cur0, cur1 = lax.fori_loop(
            0, ngroups, group, (jnp.int32(0), jnp.int32(0)))

        zero16 = lanes * 0
        pf0[pl.ds(cur0, LANES)] = plsc.load_gather(pf0, [zero16])
        pf1[pl.ds(cur1, LANES)] = plsc.load_gather(pf1, [zero16])

        def stage(k, carry):
            p2d0[k, :] = pf0[pl.ds(k * LANES, LANES)]
            p2d1[k, :] = pf1[pl.ds(k * LANES, LANES)]
            return carry

        lax.fori_loop(0, ngroups + 1, stage, 0)

        nc0 = (cur0 + (CHUNK - 1)) // CHUNK
        nc1 = (cur1 + (CHUNK - 1)) // CHUNK

        def fire0(k, carry):
            pltpu.async_copy(c0_v, out_hbm.at[p2d0.at[k]], sem)
            return carry

        def fire1(k, carry):
            pltpu.async_copy(c1_v, out_hbm.at[p2d1.at[k]], sem)
            return carry

        def fill1(j, carry):
            v1 = w_v[1, pl.ds(j * LANES, LANES)]
            for r in range(CHUNK):
                c1_v[r, pl.ds(j * LANES, LANES)] = v1
            return carry

        lax.fori_loop(0, nc0, fire0, 0)
        lax.fori_loop(0, hidden // LANES, fill1, 0)
        lax.fori_loop(0, nc1, fire1, 0)

        def drain(k, carry):
            pltpu.make_async_copy(c0_v, out_hbm.at[p2d0.at[0]], sem).wait()
            return carry

        lax.fori_loop(0, nc0 + nc1, drain, 0)

    return run


def kernel(tokentype_ids, weight):
    batch, seq = tokentype_ids.shape
    vocab, hidden = weight.shape
    num_tokens = batch * seq
    ids2 = tokentype_ids.reshape(NUM_WORKERS, num_tokens // NUM_WORKERS)
    out = _build(num_tokens, hidden)(ids2, weight)
    return out.reshape(batch, seq, hidden)
